# baseline (speedup 1.0000x reference)
import functools

import jax
import jax.numpy as jnp
from jax import lax
from jax.experimental import pallas as pl
from jax.experimental.pallas import tpu as pltpu
from jax.experimental.pallas import tpu_sc as plsc

N_ROWS = 1_000_000
COLS = 7
LANES = 16
NC, NS = 2, 16
NW = NC * NS
R = 2048
N_MAIN = (N_ROWS // 128) // (R // 128) * R
C_ALL = N_MAIN // R
BT = 8192
K_TC = 18
N_TC = K_TC * BT
C = (N_MAIN - N_TC) // R
BR = BT // 128
N_TAIL = N_ROWS - N_MAIN
GROUPS = R // LANES
UNROLL = 2
TGROUPS = N_TAIL // LANES
NMAX = -(-C // NW)
NMAX += NMAX % 2
THRESH = 0.55
INF = float("inf")


def _sc_body(pred_hbm, target_hbm, iou_hbm, ptail_hbm, ttail_hbm,
             ws_out, cnt_out,
             pbuf, tbuf, ibuf, ptb, ttb, itb, mbuf, obuf,
             sem0, sem1, sem2):
  wid = lax.axis_index("s") * NC + lax.axis_index("c")
  sems = (sem0, sem1)

  def chunk_of(i):
    return jnp.minimum(wid + NW * i, C - 1)

  def copies(slot, chunk):
    r0 = chunk * R
    sem = sems[slot]
    return [
        pltpu.make_async_copy(pred_hbm.at[:, pl.ds(r0, R)],
                              pbuf.at[slot], sem),
        pltpu.make_async_copy(target_hbm.at[:, pl.ds(r0, R)],
                              tbuf.at[slot], sem),
        pltpu.make_async_copy(iou_hbm.at[pl.ds(r0, R)],
                              ibuf.at[slot], sem),
    ]

  def issue(slot, chunk):
    for cp in copies(slot, chunk):
      cp.start()

  def drain(slot):
    for cp in copies(slot, 0):
      cp.wait()

  tail_copies = [
      pltpu.make_async_copy(ptail_hbm, ptb, sem2),
      pltpu.make_async_copy(ttail_hbm, ttb, sem2),
      pltpu.make_async_copy(iou_hbm.at[pl.ds(N_MAIN, N_TAIL)], itb, sem2),
  ]
  for cp in tail_copies:
    cp.start()

  issue(0, chunk_of(0))

  zero = jnp.zeros((LANES,), jnp.float32)

  def pair_body(p, carry):
    for s in (0, 1):
      i = 2 * p + s
      nxt = i + 1

      @pl.when(nxt < NMAX)
      def _():
        issue(s ^ 1, chunk_of(nxt))

      drain(s)
      t_eff = jnp.where(wid + NW * i <= C - 1, THRESH, INF)
      accs, cacc = carry

      def group_body(g2, c2):
        accs2, cacc2 = c2
        for u in range(UNROLL):
          g = g2 * UNROLL + u
          m16 = jnp.where(ibuf[s, pl.ds(g * LANES, LANES)] >= t_eff,
                          1.0, 0.0)
          cacc2 = cacc2 + m16
          accs3 = []
          for c in range(COLS):
            d = (pbuf[s, c, pl.ds(g * LANES, LANES)]
                 - tbuf[s, c, pl.ds(g * LANES, LANES)])
            ax = jnp.abs(d)
            mn = jnp.minimum(ax, m16)
            accs3.append(accs2[c] + mn * (ax + ax - mn))
          accs2 = tuple(accs3)
        return accs2, cacc2

      carry = lax.fori_loop(0, GROUPS // UNROLL, group_body, (accs, cacc))
    return carry

  accs, cacc = lax.fori_loop(0, NPAIRS, pair_body, ((zero,) * COLS, zero))

  ws = zero
  for c in range(COLS):
    ws = ws + jnp.float32(1.0 / 6.0 if c < 3 else 0.5) * accs[c]

  for cp in tail_copies:
    cp.wait()
  iota = lax.iota(jnp.int32, LANES)
  t_tail = jnp.where(wid == NW - 1, THRESH, INF)
  for g in range(TGROUPS):
    m16 = jnp.where(itb[pl.ds(g * LANES, LANES)] >= t_tail, 1.0, 0.0)
    cacc = cacc + m16
    mbuf[pl.ds(0, LANES)] = m16
    for j in range(COLS):
      k = iota + (g * COLS + j) * LANES
      row_in_g = (k // COLS) - g * LANES
      col = k % COLS
      wj = jnp.where(col < 3, jnp.float32(1.0 / 6.0), jnp.float32(0.5))
      off = (g * COLS + j) * LANES
      d = ptb[pl.ds(off, LANES)] - ttb[pl.ds(off, LANES)]
      ax = jnp.abs(d)
      mn = jnp.minimum(ax, plsc.load_gather(mbuf, [row_in_g]))
      ws = ws + wj * (mn * (ax + ax - mn))

  obuf[pl.ds(0, LANES)] = ws
  obuf[pl.ds(LANES, LANES)] = cacc
  pltpu.sync_copy(obuf.at[pl.ds(0, LANES)],
                  ws_out.at[pl.ds(wid * LANES, LANES)])
  pltpu.sync_copy(obuf.at[pl.ds(LANES, LANES)],
                  cnt_out.at[pl.ds(wid * LANES, LANES)])


NPAIRS = NMAX // 2


@functools.partial(
    pl.kernel,
    out_type=(jax.ShapeDtypeStruct((NW * LANES,), jnp.float32),
              jax.ShapeDtypeStruct((NW * LANES,), jnp.float32)),
    mesh=plsc.VectorSubcoreMesh(core_axis_name="c", subcore_axis_name="s",
                                num_cores=NC, num_subcores=NS),
    scratch_types=(
        pltpu.VMEM((2, COLS, R), jnp.float32),
        pltpu.VMEM((2, COLS, R), jnp.float32),
        pltpu.VMEM((2, R), jnp.float32),
        pltpu.VMEM((N_TAIL * COLS,), jnp.float32),
        pltpu.VMEM((N_TAIL * COLS,), jnp.float32),
        pltpu.VMEM((N_TAIL,), jnp.float32),
        pltpu.VMEM((LANES,), jnp.float32),
        pltpu.VMEM((2 * LANES,), jnp.float32),
        pltpu.SemaphoreType.DMA,
        pltpu.SemaphoreType.DMA,
        pltpu.SemaphoreType.DMA,
    ),
    compiler_params=pltpu.CompilerParams(needs_layout_passes=False),
)
def _sc_loss(pred_hbm, target_hbm, iou_hbm, ptail_hbm, ttail_hbm,
             ws_out, cnt_out,
             pbuf, tbuf, ibuf, ptb, ttb, itb, mbuf, obuf, sem0, sem1, sem2):
  _sc_body(pred_hbm, target_hbm, iou_hbm, ptail_hbm, ttail_hbm,
           ws_out, cnt_out,
           pbuf, tbuf, ibuf, ptb, ttb, itb, mbuf, obuf, sem0, sem1, sem2)


def _tc_kernel(pred_ref, targ_ref, iou_ref, ws_ref, cnt_ref,
               wacc_ref, cacc_ref):
  i = pl.program_id(0)

  @pl.when(i == 0)
  def _():
    wacc_ref[...] = jnp.zeros_like(wacc_ref)
    cacc_ref[...] = jnp.zeros_like(cacc_ref)

  p = pred_ref[...]
  t = targ_ref[...]
  d = p - t
  ax = jnp.abs(d)
  mn = jnp.minimum(ax, 1.0)
  kc = jnp.where(
      jax.lax.broadcasted_iota(jnp.int32, (COLS, BT), 0) < 3,
      jnp.float32(1.0 / 6.0), jnp.float32(0.5))
  s_row = jnp.sum(kc * (mn * (ax + ax - mn)), axis=0)
  s2 = s_row.reshape(BR, 128)
  m2 = jnp.where(iou_ref[...] >= THRESH, 1.0, 0.0)
  wacc_ref[...] += m2 * s2
  cacc_ref[...] += m2

  @pl.when(i == K_TC - 1)
  def _():
    wa = wacc_ref[...]
    ca = cacc_ref[...]
    ws4 = jnp.zeros((4, 128), jnp.float32)
    ca4 = jnp.zeros((4, 128), jnp.float32)
    for b in range(BR // 4):
      ws4 = ws4 + wa[4 * b:4 * b + 4]
      ca4 = ca4 + ca[4 * b:4 * b + 4]
    ws_ref[...] = ws4
    cnt_ref[...] = ca4


def _tc_loss(pred_t, target_t, iou2d):
  return pl.pallas_call(
      _tc_kernel,
      grid=(K_TC,),
      in_specs=[
          pl.BlockSpec((COLS, BT), lambda i: (0, (C * R) // BT + i)),
          pl.BlockSpec((COLS, BT), lambda i: (0, (C * R) // BT + i)),
          pl.BlockSpec((BR, 128), lambda i: ((C * R) // 128 // BR + i, 0)),
      ],
      out_specs=[
          pl.BlockSpec((4, 128), lambda i: (0, 0)),
          pl.BlockSpec((4, 128), lambda i: (0, 0)),
      ],
      out_shape=[
          jax.ShapeDtypeStruct((4, 128), jnp.float32),
          jax.ShapeDtypeStruct((4, 128), jnp.float32),
      ],
      scratch_shapes=[
          pltpu.VMEM((BR, 128), jnp.float32),
          pltpu.VMEM((BR, 128), jnp.float32),
      ],
  )(pred_t, target_t, iou2d)


def kernel(pred, target, iou):
  ptail = pred[N_MAIN:].reshape(-1)
  ttail = target[N_MAIN:].reshape(-1)
  pred_t = pred.T
  target_t = target.T
  ws, cnt = _sc_loss(pred_t, target_t, iou, ptail, ttail)
  iou2d = iou[:N_MAIN].reshape(N_MAIN // 128, 128)
  ws_tc, cnt_tc = _tc_loss(pred_t, target_t, iou2d)
  both = jnp.sum(
      jnp.stack([ws + ws_tc.reshape(-1), cnt + cnt_tc.reshape(-1)]), axis=1)
  return both[0] / both[1]


# speedup vs baseline: 1.0446x; 1.0446x over previous
import functools

import jax
import jax.numpy as jnp
from jax import lax
from jax.experimental import pallas as pl
from jax.experimental.pallas import tpu as pltpu
from jax.experimental.pallas import tpu_sc as plsc

N_ROWS = 1_000_000
COLS = 7
LANES = 16
NC, NS = 2, 16
NW = NC * NS
R = 2048
N_MAIN = (N_ROWS // 128) // (R // 128) * R
C_ALL = N_MAIN // R
BT = 8192
K_TC = 10
N_TC = K_TC * BT
C = (N_MAIN - N_TC) // R
BR = BT // 128
N_TAIL = N_ROWS - N_MAIN
GROUPS = R // LANES
UNROLL = 2
TGROUPS = N_TAIL // LANES
NMAX = -(-C // NW)
NMAX += NMAX % 2
THRESH = 0.55
INF = float("inf")


def _sc_body(pred_hbm, target_hbm, iou_hbm, ptail_hbm, ttail_hbm,
             ws_out, cnt_out,
             pbuf, tbuf, ibuf, ptb, ttb, itb, mbuf, obuf,
             sem0, sem1, sem2):
  wid = lax.axis_index("s") * NC + lax.axis_index("c")
  sems = (sem0, sem1)

  def chunk_of(i):
    return jnp.minimum(wid + NW * i, C - 1)

  def copies(slot, chunk):
    r0 = chunk * R
    sem = sems[slot]
    return [
        pltpu.make_async_copy(pred_hbm.at[:, pl.ds(r0, R)],
                              pbuf.at[slot], sem),
        pltpu.make_async_copy(target_hbm.at[:, pl.ds(r0, R)],
                              tbuf.at[slot], sem),
        pltpu.make_async_copy(iou_hbm.at[pl.ds(r0, R)],
                              ibuf.at[slot], sem),
    ]

  def issue(slot, chunk):
    for cp in copies(slot, chunk):
      cp.start()

  def drain(slot):
    for cp in copies(slot, 0):
      cp.wait()

  tail_copies = [
      pltpu.make_async_copy(ptail_hbm, ptb, sem2),
      pltpu.make_async_copy(ttail_hbm, ttb, sem2),
      pltpu.make_async_copy(iou_hbm.at[pl.ds(N_MAIN, N_TAIL)], itb, sem2),
  ]
  for cp in tail_copies:
    cp.start()

  issue(0, chunk_of(0))

  zero = jnp.zeros((LANES,), jnp.float32)

  def pair_body(p, carry):
    for s in (0, 1):
      i = 2 * p + s
      nxt = i + 1

      @pl.when(nxt < NMAX)
      def _():
        issue(s ^ 1, chunk_of(nxt))

      drain(s)
      t_eff = jnp.where(wid + NW * i <= C - 1, THRESH, INF)
      accs, cacc = carry

      def group_body(g2, c2):
        accs2, cacc2 = c2
        for u in range(UNROLL):
          g = g2 * UNROLL + u
          m16 = jnp.where(ibuf[s, pl.ds(g * LANES, LANES)] >= t_eff,
                          1.0, 0.0)
          cacc2 = cacc2 + m16
          accs3 = []
          for c in range(COLS):
            d = (pbuf[s, c, pl.ds(g * LANES, LANES)]
                 - tbuf[s, c, pl.ds(g * LANES, LANES)])
            ax = jnp.abs(d)
            mn = jnp.minimum(ax, m16)
            accs3.append(accs2[c] + mn * (ax + ax - mn))
          accs2 = tuple(accs3)
        return accs2, cacc2

      carry = lax.fori_loop(0, GROUPS // UNROLL, group_body, (accs, cacc))
    return carry

  accs, cacc = lax.fori_loop(0, NPAIRS, pair_body, ((zero,) * COLS, zero))

  ws = zero
  for c in range(COLS):
    ws = ws + jnp.float32(1.0 / 6.0 if c < 3 else 0.5) * accs[c]

  for cp in tail_copies:
    cp.wait()
  iota = lax.iota(jnp.int32, LANES)
  t_tail = jnp.where(wid == NW - 1, THRESH, INF)
  for g in range(TGROUPS):
    m16 = jnp.where(itb[pl.ds(g * LANES, LANES)] >= t_tail, 1.0, 0.0)
    cacc = cacc + m16
    mbuf[pl.ds(0, LANES)] = m16
    for j in range(COLS):
      k = iota + (g * COLS + j) * LANES
      row_in_g = (k // COLS) - g * LANES
      col = k % COLS
      wj = jnp.where(col < 3, jnp.float32(1.0 / 6.0), jnp.float32(0.5))
      off = (g * COLS + j) * LANES
      d = ptb[pl.ds(off, LANES)] - ttb[pl.ds(off, LANES)]
      ax = jnp.abs(d)
      mn = jnp.minimum(ax, plsc.load_gather(mbuf, [row_in_g]))
      ws = ws + wj * (mn * (ax + ax - mn))

  obuf[pl.ds(0, LANES)] = ws
  obuf[pl.ds(LANES, LANES)] = cacc
  pltpu.sync_copy(obuf.at[pl.ds(0, LANES)],
                  ws_out.at[pl.ds(wid * LANES, LANES)])
  pltpu.sync_copy(obuf.at[pl.ds(LANES, LANES)],
                  cnt_out.at[pl.ds(wid * LANES, LANES)])


NPAIRS = NMAX // 2


@functools.partial(
    pl.kernel,
    out_type=(jax.ShapeDtypeStruct((NW * LANES,), jnp.float32),
              jax.ShapeDtypeStruct((NW * LANES,), jnp.float32)),
    mesh=plsc.VectorSubcoreMesh(core_axis_name="c", subcore_axis_name="s",
                                num_cores=NC, num_subcores=NS),
    scratch_types=(
        pltpu.VMEM((2, COLS, R), jnp.float32),
        pltpu.VMEM((2, COLS, R), jnp.float32),
        pltpu.VMEM((2, R), jnp.float32),
        pltpu.VMEM((N_TAIL * COLS,), jnp.float32),
        pltpu.VMEM((N_TAIL * COLS,), jnp.float32),
        pltpu.VMEM((N_TAIL,), jnp.float32),
        pltpu.VMEM((LANES,), jnp.float32),
        pltpu.VMEM((2 * LANES,), jnp.float32),
        pltpu.SemaphoreType.DMA,
        pltpu.SemaphoreType.DMA,
        pltpu.SemaphoreType.DMA,
    ),
    compiler_params=pltpu.CompilerParams(needs_layout_passes=False),
)
def _sc_loss(pred_hbm, target_hbm, iou_hbm, ptail_hbm, ttail_hbm,
             ws_out, cnt_out,
             pbuf, tbuf, ibuf, ptb, ttb, itb, mbuf, obuf, sem0, sem1, sem2):
  _sc_body(pred_hbm, target_hbm, iou_hbm, ptail_hbm, ttail_hbm,
           ws_out, cnt_out,
           pbuf, tbuf, ibuf, ptb, ttb, itb, mbuf, obuf, sem0, sem1, sem2)


def _tc_kernel(pred_ref, targ_ref, iou_ref, ws_ref, cnt_ref,
               wacc_ref, cacc_ref):
  i = pl.program_id(0)

  @pl.when(i == 0)
  def _():
    wacc_ref[...] = jnp.zeros_like(wacc_ref)
    cacc_ref[...] = jnp.zeros_like(cacc_ref)

  p = pred_ref[...]
  t = targ_ref[...]
  d = p - t
  ax = jnp.abs(d)
  mn = jnp.minimum(ax, 1.0)
  kc = jnp.where(
      jax.lax.broadcasted_iota(jnp.int32, (COLS, BT), 0) < 3,
      jnp.float32(1.0 / 6.0), jnp.float32(0.5))
  s_row = jnp.sum(kc * (mn * (ax + ax - mn)), axis=0)
  s2 = s_row.reshape(BR, 128)
  m2 = jnp.where(iou_ref[...] >= THRESH, 1.0, 0.0)
  wacc_ref[...] += m2 * s2
  cacc_ref[...] += m2

  @pl.when(i == K_TC - 1)
  def _():
    wa = wacc_ref[...]
    ca = cacc_ref[...]
    ws4 = jnp.zeros((4, 128), jnp.float32)
    ca4 = jnp.zeros((4, 128), jnp.float32)
    for b in range(BR // 4):
      ws4 = ws4 + wa[4 * b:4 * b + 4]
      ca4 = ca4 + ca[4 * b:4 * b + 4]
    ws_ref[...] = ws4
    cnt_ref[...] = ca4


def _tc_loss(pred_t, target_t, iou2d):
  return pl.pallas_call(
      _tc_kernel,
      grid=(K_TC,),
      in_specs=[
          pl.BlockSpec((COLS, BT), lambda i: (0, (C * R) // BT + i)),
          pl.BlockSpec((COLS, BT), lambda i: (0, (C * R) // BT + i)),
          pl.BlockSpec((BR, 128), lambda i: ((C * R) // 128 // BR + i, 0)),
      ],
      out_specs=[
          pl.BlockSpec((4, 128), lambda i: (0, 0)),
          pl.BlockSpec((4, 128), lambda i: (0, 0)),
      ],
      out_shape=[
          jax.ShapeDtypeStruct((4, 128), jnp.float32),
          jax.ShapeDtypeStruct((4, 128), jnp.float32),
      ],
      scratch_shapes=[
          pltpu.VMEM((BR, 128), jnp.float32),
          pltpu.VMEM((BR, 128), jnp.float32),
      ],
  )(pred_t, target_t, iou2d)


def kernel(pred, target, iou):
  ptail = pred[N_MAIN:].reshape(-1)
  ttail = target[N_MAIN:].reshape(-1)
  pred_t = pred.T
  target_t = target.T
  ws, cnt = _sc_loss(pred_t, target_t, iou, ptail, ttail)
  iou2d = iou[:N_MAIN].reshape(N_MAIN // 128, 128)
  ws_tc, cnt_tc = _tc_loss(pred_t, target_t, iou2d)
  both = jnp.sum(
      jnp.stack([ws + ws_tc.reshape(-1), cnt + cnt_tc.reshape(-1)]), axis=1)
  return both[0] / both[1]


# speedup vs baseline: 1.1039x; 1.0568x over previous
import functools

import jax
import jax.numpy as jnp
from jax import lax
from jax.experimental import pallas as pl
from jax.experimental.pallas import tpu as pltpu
from jax.experimental.pallas import tpu_sc as plsc

N_ROWS = 1_000_000
COLS = 7
LANES = 16
NC, NS = 2, 16
NW = NC * NS
R = 2048
N_MAIN = (N_ROWS // 128) // (R // 128) * R
C_ALL = N_MAIN // R
BT = 8192
K_TC = 26
N_TC = K_TC * BT
C = (N_MAIN - N_TC) // R
BR = BT // 128
N_TAIL = N_ROWS - N_MAIN
GROUPS = R // LANES
UNROLL = 2
TGROUPS = N_TAIL // LANES
NMAX = -(-C // NW)
NMAX += NMAX % 2
THRESH = 0.55
INF = float("inf")


def _sc_body(pred_hbm, target_hbm, iou_hbm, ptail_hbm, ttail_hbm,
             ws_out, cnt_out,
             pbuf, tbuf, ibuf, ptb, ttb, itb, mbuf, obuf,
             sem0, sem1, sem2):
  wid = lax.axis_index("s") * NC + lax.axis_index("c")
  sems = (sem0, sem1)

  def chunk_of(i):
    return jnp.minimum(wid + NW * i, C - 1)

  def copies(slot, chunk):
    r0 = chunk * R
    sem = sems[slot]
    return [
        pltpu.make_async_copy(pred_hbm.at[:, pl.ds(r0, R)],
                              pbuf.at[slot], sem),
        pltpu.make_async_copy(target_hbm.at[:, pl.ds(r0, R)],
                              tbuf.at[slot], sem),
        pltpu.make_async_copy(iou_hbm.at[pl.ds(r0, R)],
                              ibuf.at[slot], sem),
    ]

  def issue(slot, chunk):
    for cp in copies(slot, chunk):
      cp.start()

  def drain(slot):
    for cp in copies(slot, 0):
      cp.wait()

  tail_copies = [
      pltpu.make_async_copy(ptail_hbm, ptb, sem2),
      pltpu.make_async_copy(ttail_hbm, ttb, sem2),
      pltpu.make_async_copy(iou_hbm.at[pl.ds(N_MAIN, N_TAIL)], itb, sem2),
  ]
  for cp in tail_copies:
    cp.start()

  issue(0, chunk_of(0))

  zero = jnp.zeros((LANES,), jnp.float32)

  def pair_body(p, carry):
    for s in (0, 1):
      i = 2 * p + s
      nxt = i + 1

      @pl.when(nxt < NMAX)
      def _():
        issue(s ^ 1, chunk_of(nxt))

      drain(s)
      t_eff = jnp.where(wid + NW * i <= C - 1, THRESH, INF)
      accs, cacc = carry

      def group_body(g2, c2):
        accs2, cacc2 = c2
        for u in range(UNROLL):
          g = g2 * UNROLL + u
          m16 = jnp.where(ibuf[s, pl.ds(g * LANES, LANES)] >= t_eff,
                          1.0, 0.0)
          cacc2 = cacc2 + m16
          accs3 = []
          for c in range(COLS):
            d = (pbuf[s, c, pl.ds(g * LANES, LANES)]
                 - tbuf[s, c, pl.ds(g * LANES, LANES)])
            ax = jnp.abs(d)
            mn = jnp.minimum(ax, m16)
            accs3.append(accs2[c] + mn * (ax + ax - mn))
          accs2 = tuple(accs3)
        return accs2, cacc2

      carry = lax.fori_loop(0, GROUPS // UNROLL, group_body, (accs, cacc))
    return carry

  accs, cacc = lax.fori_loop(0, NPAIRS, pair_body, ((zero,) * COLS, zero))

  ws = zero
  for c in range(COLS):
    ws = ws + jnp.float32(1.0 / 6.0 if c < 3 else 0.5) * accs[c]

  for cp in tail_copies:
    cp.wait()
  iota = lax.iota(jnp.int32, LANES)
  t_tail = jnp.where(wid == NW - 1, THRESH, INF)
  for g in range(TGROUPS):
    m16 = jnp.where(itb[pl.ds(g * LANES, LANES)] >= t_tail, 1.0, 0.0)
    cacc = cacc + m16
    mbuf[pl.ds(0, LANES)] = m16
    for j in range(COLS):
      k = iota + (g * COLS + j) * LANES
      row_in_g = (k // COLS) - g * LANES
      col = k % COLS
      wj = jnp.where(col < 3, jnp.float32(1.0 / 6.0), jnp.float32(0.5))
      off = (g * COLS + j) * LANES
      d = ptb[pl.ds(off, LANES)] - ttb[pl.ds(off, LANES)]
      ax = jnp.abs(d)
      mn = jnp.minimum(ax, plsc.load_gather(mbuf, [row_in_g]))
      ws = ws + wj * (mn * (ax + ax - mn))

  obuf[pl.ds(0, LANES)] = ws
  obuf[pl.ds(LANES, LANES)] = cacc
  pltpu.sync_copy(obuf.at[pl.ds(0, LANES)],
                  ws_out.at[pl.ds(wid * LANES, LANES)])
  pltpu.sync_copy(obuf.at[pl.ds(LANES, LANES)],
                  cnt_out.at[pl.ds(wid * LANES, LANES)])


NPAIRS = NMAX // 2


@functools.partial(
    pl.kernel,
    out_type=(jax.ShapeDtypeStruct((NW * LANES,), jnp.float32),
              jax.ShapeDtypeStruct((NW * LANES,), jnp.float32)),
    mesh=plsc.VectorSubcoreMesh(core_axis_name="c", subcore_axis_name="s",
                                num_cores=NC, num_subcores=NS),
    scratch_types=(
        pltpu.VMEM((2, COLS, R), jnp.float32),
        pltpu.VMEM((2, COLS, R), jnp.float32),
        pltpu.VMEM((2, R), jnp.float32),
        pltpu.VMEM((N_TAIL * COLS,), jnp.float32),
        pltpu.VMEM((N_TAIL * COLS,), jnp.float32),
        pltpu.VMEM((N_TAIL,), jnp.float32),
        pltpu.VMEM((LANES,), jnp.float32),
        pltpu.VMEM((2 * LANES,), jnp.float32),
        pltpu.SemaphoreType.DMA,
        pltpu.SemaphoreType.DMA,
        pltpu.SemaphoreType.DMA,
    ),
    compiler_params=pltpu.CompilerParams(needs_layout_passes=False),
)
def _sc_loss(pred_hbm, target_hbm, iou_hbm, ptail_hbm, ttail_hbm,
             ws_out, cnt_out,
             pbuf, tbuf, ibuf, ptb, ttb, itb, mbuf, obuf, sem0, sem1, sem2):
  _sc_body(pred_hbm, target_hbm, iou_hbm, ptail_hbm, ttail_hbm,
           ws_out, cnt_out,
           pbuf, tbuf, ibuf, ptb, ttb, itb, mbuf, obuf, sem0, sem1, sem2)


def _tc_kernel(pred_ref, targ_ref, iou_ref, ws_ref, cnt_ref,
               wacc_ref, cacc_ref):
  i = pl.program_id(0)

  @pl.when(i == 0)
  def _():
    wacc_ref[...] = jnp.zeros_like(wacc_ref)
    cacc_ref[...] = jnp.zeros_like(cacc_ref)

  p = pred_ref[...]
  t = targ_ref[...]
  d = p - t
  ax = jnp.abs(d)
  mn = jnp.minimum(ax, 1.0)
  kc = jnp.where(
      jax.lax.broadcasted_iota(jnp.int32, (COLS, BT), 0) < 3,
      jnp.float32(1.0 / 6.0), jnp.float32(0.5))
  s_row = jnp.sum(kc * (mn * (ax + ax - mn)), axis=0)
  s2 = s_row.reshape(BR, 128)
  m2 = jnp.where(iou_ref[...] >= THRESH, 1.0, 0.0)
  wacc_ref[...] += m2 * s2
  cacc_ref[...] += m2

  @pl.when(i == K_TC - 1)
  def _():
    wa = wacc_ref[...]
    ca = cacc_ref[...]
    ws4 = jnp.zeros((4, 128), jnp.float32)
    ca4 = jnp.zeros((4, 128), jnp.float32)
    for b in range(BR // 4):
      ws4 = ws4 + wa[4 * b:4 * b + 4]
      ca4 = ca4 + ca[4 * b:4 * b + 4]
    ws_ref[...] = ws4
    cnt_ref[...] = ca4


def _tc_loss(pred_t, target_t, iou2d):
  return pl.pallas_call(
      _tc_kernel,
      grid=(K_TC,),
      in_specs=[
          pl.BlockSpec((COLS, BT), lambda i: (0, (C * R) // BT + i)),
          pl.BlockSpec((COLS, BT), lambda i: (0, (C * R) // BT + i)),
          pl.BlockSpec((BR, 128), lambda i: ((C * R) // 128 // BR + i, 0)),
      ],
      out_specs=[
          pl.BlockSpec((4, 128), lambda i: (0, 0)),
          pl.BlockSpec((4, 128), lambda i: (0, 0)),
      ],
      out_shape=[
          jax.ShapeDtypeStruct((4, 128), jnp.float32),
          jax.ShapeDtypeStruct((4, 128), jnp.float32),
      ],
      scratch_shapes=[
          pltpu.VMEM((BR, 128), jnp.float32),
          pltpu.VMEM((BR, 128), jnp.float32),
      ],
  )(pred_t, target_t, iou2d)


def kernel(pred, target, iou):
  ptail = pred[N_MAIN:].reshape(-1)
  ttail = target[N_MAIN:].reshape(-1)
  pred_t = pred.T
  target_t = target.T
  ws, cnt = _sc_loss(pred_t, target_t, iou, ptail, ttail)
  iou2d = iou[:N_MAIN].reshape(N_MAIN // 128, 128)
  ws_tc, cnt_tc = _tc_loss(pred_t, target_t, iou2d)
  both = jnp.sum(
      jnp.stack([ws + ws_tc.reshape(-1), cnt + cnt_tc.reshape(-1)]), axis=1)
  return both[0] / both[1]
